# Initial kernel scaffold; baseline (speedup 1.0000x reference)
import jax
import jax.numpy as jnp
from jax.experimental import pallas as pl


def kernel(x, edge_index, fc1_W, fc1_b, gat_W, att_src, att_dst, gat_b):
    raise NotImplementedError("write your pallas kernel here")


# speedup vs baseline: 17.3515x; 17.3515x over previous
import functools

import jax
import jax.numpy as jnp
from jax import lax
from jax.experimental import pallas as pl
from jax.experimental.pallas import tpu as pltpu
from jax.experimental.pallas import tpu_sc as plsc

N = 10000
E = 320000
HID = 128
OUT = 128

NC = 2
NS = 16
L = 16
NW = NC * NS
EPW = E // NW
K = 80
NCH = EPW // K
NRC = N // K
RRK = (NRC + NS - 1) // NS

BLK = 1000
G = N // BLK


def _dense_body(x_ref, w1_ref, b1_ref, w2_ref, as_w_ref, ad_w_ref,
                z_ref, asrc_ref, adst_ref, pmax_ref):
  x = x_ref[...]
  h = lax.dot_general(x, w1_ref[...], (((1,), (1,)), ((), ())),
                      preferred_element_type=jnp.float32)
  h = jnp.maximum(h + b1_ref[...], 0.0)
  z = jnp.dot(h, w2_ref[...], preferred_element_type=jnp.float32)
  z_ref[...] = z
  a_s = jnp.sum(z * as_w_ref[...], axis=1, keepdims=True)
  a_d = jnp.sum(z * ad_w_ref[...], axis=1, keepdims=True)
  asrc_ref[...] = a_s
  adst_ref[...] = a_d
  i = pl.program_id(0)
  pmax_ref[pl.ds(i, 1), :] = jnp.max(a_s).reshape(1, 1)


_dense = pl.pallas_call(
    _dense_body,
    grid=(G,),
    in_specs=[
        pl.BlockSpec((BLK, HID), lambda i: (i, 0)),
        pl.BlockSpec((HID, HID), lambda i: (0, 0)),
        pl.BlockSpec((1, HID), lambda i: (0, 0)),
        pl.BlockSpec((HID, OUT), lambda i: (0, 0)),
        pl.BlockSpec((1, OUT), lambda i: (0, 0)),
        pl.BlockSpec((1, OUT), lambda i: (0, 0)),
    ],
    out_specs=[
        pl.BlockSpec((BLK, OUT), lambda i: (i, 0)),
        pl.BlockSpec((BLK, 1), lambda i: (i, 0)),
        pl.BlockSpec((BLK, 1), lambda i: (i, 0)),
        pl.BlockSpec((G, 1), lambda i: (0, 0)),
    ],
    out_shape=[
        jax.ShapeDtypeStruct((N, OUT), jnp.float32),
        jax.ShapeDtypeStruct((N, 1), jnp.float32),
        jax.ShapeDtypeStruct((N, 1), jnp.float32),
        jax.ShapeDtypeStruct((G, 1), jnp.float32),
    ],
)


def _sc_body(src_hbm, dst_hbm, asrc_hbm, adst_hbm, maxa_hbm, z_hbm,
             s1_hbm, s0_hbm,
             asrc_v, adst_v, maxa_v, sidx_v, didx_v, w_v, rows_v,
             s0_v, s1_sh, sem):
  c = lax.axis_index("c")
  s = lax.axis_index("s")
  wid = c * NS + s

  pltpu.sync_copy(asrc_hbm, asrc_v)
  pltpu.sync_copy(adst_hbm, adst_v)
  pltpu.sync_copy(maxa_hbm, maxa_v)

  zv = jnp.zeros((L,), jnp.float32)

  def _zrow(r, carry):
    for q in range(OUT // L):
      rows_v[r, pl.ds(q * L, L)] = zv
    return carry

  lax.fori_loop(0, K, _zrow, 0)

  def _zs0(i, carry):
    s0_v[pl.ds(pl.multiple_of(i * L, 8), L)] = zv
    return carry

  lax.fori_loop(0, N // L, _zs0, 0)

  for k in range(RRK):
    j = s + NS * k

    @pl.when(j < NRC)
    def _zero_s1():
      pltpu.sync_copy(rows_v, s1_sh.at[pl.ds(j * K, K)])

  plsc.subcore_barrier()

  maxa = maxa_v[...]

  def _chunk(i, carry):
    base = pl.multiple_of(wid * EPW + i * K, 8)
    pltpu.sync_copy(src_hbm.at[pl.ds(base, K)], sidx_v)
    pltpu.sync_copy(dst_hbm.at[pl.ds(base, K)], didx_v)
    gat = pltpu.async_copy(z_hbm.at[sidx_v], rows_v, sem)
    for j in range(K // L):
      si = sidx_v[pl.ds(j * L, L)]
      di = didx_v[pl.ds(j * L, L)]
      gs = plsc.load_gather(asrc_v, (si,))
      gd = plsc.load_gather(adst_v, (di,))
      e = gs + gd
      e = jnp.where(e >= 0.0, e, 0.2 * e)
      m = maxa + gd
      m = jnp.where(m >= 0.0, m, 0.2 * m)
      w = jnp.exp(e - m)
      w_v[pl.ds(j * L, L)] = w
      plsc.addupdate_scatter(s0_v, (di,), w)
    gat.wait()

    def _scale(g, carry2):
      wv = w_v[pl.ds(pl.multiple_of(g * L, 8), L)]
      for t in range(L):
        wj = wv[t]
        j = g * L + t
        for q in range(OUT // L):
          rows_v[j, pl.ds(q * L, L)] = rows_v[j, pl.ds(q * L, L)] * wj
      return carry2

    lax.fori_loop(0, K // L, _scale, 0)
    pltpu.sync_copy(rows_v, s1_sh.at[didx_v], add=True)
    return carry

  lax.fori_loop(0, NCH, _chunk, 0)
  plsc.subcore_barrier()

  for k in range(RRK):
    j = s + NS * k

    @pl.when(j < NRC)
    def _out_s1():
      r0 = j * K
      pltpu.sync_copy(s1_sh.at[pl.ds(r0, K)], rows_v)
      pltpu.sync_copy(rows_v, s1_hbm.at[c, pl.ds(r0, K)])

  pltpu.sync_copy(s0_v, s0_hbm.at[pl.ds(pl.multiple_of(wid * N, 8), N)])


_edge = functools.partial(
    pl.kernel,
    out_type=(
        jax.ShapeDtypeStruct((NC, N, OUT), jnp.float32),
        jax.ShapeDtypeStruct((NW * N,), jnp.float32),
    ),
    mesh=plsc.VectorSubcoreMesh(core_axis_name="c", subcore_axis_name="s"),
    scratch_types=[
        pltpu.VMEM((N,), jnp.float32),
        pltpu.VMEM((N,), jnp.float32),
        pltpu.VMEM((L,), jnp.float32),
        pltpu.VMEM((K,), jnp.int32),
        pltpu.VMEM((K,), jnp.int32),
        pltpu.VMEM((K,), jnp.float32),
        pltpu.VMEM((K, OUT), jnp.float32),
        pltpu.VMEM((N,), jnp.float32),
        pltpu.VMEM_SHARED((N, OUT), jnp.float32),
        pltpu.SemaphoreType.DMA,
    ],
    compiler_params=pltpu.CompilerParams(needs_layout_passes=False),
)(_sc_body)


def _combine_body(s1_ref, s0_ref, b_ref, o_ref):
  s1 = s1_ref[0] + s1_ref[1]
  s0 = jnp.sum(s0_ref[...], axis=0)
  o_ref[...] = jnp.maximum(s1 / (s0 + 1e-16) + b_ref[...], 0.0)


_combine = pl.pallas_call(
    _combine_body,
    grid=(G,),
    in_specs=[
        pl.BlockSpec((NC, BLK, OUT), lambda i: (0, i, 0)),
        pl.BlockSpec((NW, BLK, 1), lambda i: (0, i, 0)),
        pl.BlockSpec((1, OUT), lambda i: (0, 0)),
    ],
    out_specs=pl.BlockSpec((BLK, OUT), lambda i: (i, 0)),
    out_shape=jax.ShapeDtypeStruct((N, OUT), jnp.float32),
)


@jax.jit
def kernel(x, edge_index, fc1_W, fc1_b, gat_W, att_src, att_dst, gat_b):
  z, a_s, a_d, pmax = _dense(
      x, fc1_W, fc1_b.reshape(1, HID), gat_W,
      att_src.reshape(1, OUT), att_dst.reshape(1, OUT))
  maxa16 = jnp.full((L,), jnp.max(pmax), jnp.float32)
  src = edge_index[0]
  dst = edge_index[1]
  s1p, s0p = _edge(src, dst, a_s.reshape(N), a_d.reshape(N), maxa16, z)
  return _combine(s1p, s0p.reshape(NW, N, 1), gat_b.reshape(1, OUT))


# speedup vs baseline: 43.5068x; 2.5074x over previous
import functools

import jax
import jax.numpy as jnp
from jax import lax
from jax.experimental import pallas as pl
from jax.experimental.pallas import tpu as pltpu
from jax.experimental.pallas import tpu_sc as plsc

N = 10000
E = 320000
HID = 128
OUT = 128

NC = 2
NS = 16
L = 16
NW = NC * NS
EPW = E // NW
K = 80
NCH = EPW // K
CPG = 25
GPT = NCH // CPG
NRC = N // K
RRK = (NRC + NS - 1) // NS

BLK = 1000
G = N // BLK


def _dense_body(x_ref, w1_ref, b1_ref, w2_ref, as_w_ref, ad_w_ref,
                z_ref, asrc_ref, adst_ref, pmax_ref):
  x = x_ref[...]
  h = lax.dot_general(x, w1_ref[...], (((1,), (1,)), ((), ())),
                      preferred_element_type=jnp.float32)
  h = jnp.maximum(h + b1_ref[...], 0.0)
  z = jnp.dot(h, w2_ref[...], preferred_element_type=jnp.float32)
  z_ref[...] = z
  a_s = jnp.sum(z * as_w_ref[...], axis=1, keepdims=True)
  a_d = jnp.sum(z * ad_w_ref[...], axis=1, keepdims=True)
  asrc_ref[...] = a_s
  adst_ref[...] = a_d
  i = pl.program_id(0)
  pmax_ref[pl.ds(i, 1), :] = jnp.max(a_s).reshape(1, 1)


_dense = pl.pallas_call(
    _dense_body,
    grid=(G,),
    in_specs=[
        pl.BlockSpec((BLK, HID), lambda i: (i, 0)),
        pl.BlockSpec((HID, HID), lambda i: (0, 0)),
        pl.BlockSpec((1, HID), lambda i: (0, 0)),
        pl.BlockSpec((HID, OUT), lambda i: (0, 0)),
        pl.BlockSpec((1, OUT), lambda i: (0, 0)),
        pl.BlockSpec((1, OUT), lambda i: (0, 0)),
    ],
    out_specs=[
        pl.BlockSpec((BLK, OUT), lambda i: (i, 0)),
        pl.BlockSpec((BLK, 1), lambda i: (i, 0)),
        pl.BlockSpec((BLK, 1), lambda i: (i, 0)),
        pl.BlockSpec((G, 1), lambda i: (0, 0)),
    ],
    out_shape=[
        jax.ShapeDtypeStruct((N, OUT), jnp.float32),
        jax.ShapeDtypeStruct((N, 1), jnp.float32),
        jax.ShapeDtypeStruct((N, 1), jnp.float32),
        jax.ShapeDtypeStruct((G, 1), jnp.float32),
    ],
)


def _sc_body(src_hbm, dst_hbm, asrc_hbm, adst_hbm, maxa_hbm, z_hbm,
             s1_hbm, s0_hbm,
             asrc_v, adst_v, maxa_v, sidx_g, didx_g, w0, w1, rows0, rows1,
             s0_sh, s1_sh, sem0, sem1):
  c = lax.axis_index("c")
  s = lax.axis_index("s")
  wid = c * NS + s

  pltpu.sync_copy(asrc_hbm, asrc_v)
  pltpu.sync_copy(adst_hbm, adst_v)
  pltpu.sync_copy(maxa_hbm, maxa_v)

  zv = jnp.zeros((L,), jnp.float32)

  for q in range(K // L):
    w0[pl.ds(q * L, L)] = zv

  def _zrow(r, carry):
    for q in range(OUT // L):
      rows0[r, pl.ds(q * L, L)] = zv
    return carry

  lax.fori_loop(0, K, _zrow, 0)

  for k in range(RRK):
    j = s + NS * k

    @pl.when(j < NRC)
    def _zero_acc():
      pltpu.sync_copy(rows0, s1_sh.at[pl.ds(j * K, K)])
      pltpu.sync_copy(w0, s0_sh.at[pl.ds(j * K, K)])

  plsc.subcore_barrier()

  maxa = maxa_v[...]

  def _do_chunk(j, rows_b, w_b, sem_b, rows_n, sem_n):
    for q in range(K // L):
      si = sidx_g[j, pl.ds(q * L, L)]
      di = didx_g[j, pl.ds(q * L, L)]
      gs = plsc.load_gather(asrc_v, (si,))
      gd = plsc.load_gather(adst_v, (di,))
      e = gs + gd
      e = jnp.where(e >= 0.0, e, 0.2 * e)
      m = maxa + gd
      m = jnp.where(m >= 0.0, m, 0.2 * m)
      w_b[pl.ds(q * L, L)] = jnp.exp(e - m)
    pltpu.make_async_copy(z_hbm.at[sidx_g.at[j]], rows_b, sem_b).wait()

    @pl.when(j + 1 < CPG)
    def _fire_next():
      pltpu.async_copy(z_hbm.at[sidx_g.at[j + 1]], rows_n, sem_n)

    def _scale(g, carry2):
      wv = w_b[pl.ds(pl.multiple_of(g * L, 8), L)]
      for t in range(L):
        wj = wv[t]
        r = g * L + t
        for q in range(OUT // L):
          rows_b[r, pl.ds(q * L, L)] = rows_b[r, pl.ds(q * L, L)] * wj
      return carry2

    lax.fori_loop(0, K // L, _scale, 0)
    pltpu.sync_copy(rows_b, s1_sh.at[didx_g.at[j]], add=True)
    pltpu.sync_copy(w_b, s0_sh.at[didx_g.at[j]], add=True)

  def _group(g, carry):
    gb = wid * GPT + g
    pltpu.sync_copy(src_hbm.at[gb], sidx_g)
    pltpu.sync_copy(dst_hbm.at[gb], didx_g)
    pltpu.async_copy(z_hbm.at[sidx_g.at[0]], rows0, sem0)

    def _pair(t, carry2):
      _do_chunk(2 * t, rows0, w0, sem0, rows1, sem1)

      @pl.when(2 * t + 1 < CPG)
      def _odd():
        _do_chunk(2 * t + 1, rows1, w1, sem1, rows0, sem0)

      return carry2

    lax.fori_loop(0, (CPG + 1) // 2, _pair, 0)
    return carry

  lax.fori_loop(0, GPT, _group, 0)
  plsc.subcore_barrier()

  for k in range(RRK):
    j = s + NS * k

    @pl.when(j < NRC)
    def _out_acc():
      r0 = pl.multiple_of(j * K, 8)
      pltpu.sync_copy(s1_sh.at[pl.ds(r0, K)], rows0)
      pltpu.sync_copy(rows0, s1_hbm.at[c, pl.ds(r0, K)])
      pltpu.sync_copy(s0_sh.at[pl.ds(r0, K)], w0)
      pltpu.sync_copy(w0, s0_hbm.at[pl.ds(pl.multiple_of(c * N + r0, 8), K)])


_edge = functools.partial(
    pl.kernel,
    out_type=(
        jax.ShapeDtypeStruct((NC, N, OUT), jnp.float32),
        jax.ShapeDtypeStruct((NC * N,), jnp.float32),
    ),
    mesh=plsc.VectorSubcoreMesh(core_axis_name="c", subcore_axis_name="s"),
    scratch_types=[
        pltpu.VMEM((N,), jnp.float32),
        pltpu.VMEM((N,), jnp.float32),
        pltpu.VMEM((L,), jnp.float32),
        pltpu.VMEM((CPG, K), jnp.int32),
        pltpu.VMEM((CPG, K), jnp.int32),
        pltpu.VMEM((K,), jnp.float32),
        pltpu.VMEM((K,), jnp.float32),
        pltpu.VMEM((K, OUT), jnp.float32),
        pltpu.VMEM((K, OUT), jnp.float32),
        pltpu.VMEM_SHARED((N,), jnp.float32),
        pltpu.VMEM_SHARED((N, OUT), jnp.float32),
        pltpu.SemaphoreType.DMA,
        pltpu.SemaphoreType.DMA,
    ],
    compiler_params=pltpu.CompilerParams(needs_layout_passes=False),
)(_sc_body)


def _combine_body(s1_ref, s0_ref, b_ref, o_ref):
  s1 = s1_ref[0] + s1_ref[1]
  s0 = jnp.sum(s0_ref[...], axis=0)
  o_ref[...] = jnp.maximum(s1 / (s0 + 1e-16) + b_ref[...], 0.0)


_combine = pl.pallas_call(
    _combine_body,
    grid=(G,),
    in_specs=[
        pl.BlockSpec((NC, BLK, OUT), lambda i: (0, i, 0)),
        pl.BlockSpec((NC, BLK, 1), lambda i: (0, i, 0)),
        pl.BlockSpec((1, OUT), lambda i: (0, 0)),
    ],
    out_specs=pl.BlockSpec((BLK, OUT), lambda i: (i, 0)),
    out_shape=jax.ShapeDtypeStruct((N, OUT), jnp.float32),
)


@jax.jit
def kernel(x, edge_index, fc1_W, fc1_b, gat_W, att_src, att_dst, gat_b):
  z, a_s, a_d, pmax = _dense(
      x, fc1_W, fc1_b.reshape(1, HID), gat_W,
      att_src.reshape(1, OUT), att_dst.reshape(1, OUT))
  maxa16 = jnp.full((L,), jnp.max(pmax), jnp.float32)
  src = edge_index[0].reshape(NW * GPT, CPG, K)
  dst = edge_index[1].reshape(NW * GPT, CPG, K)
  s1p, s0p = _edge(src, dst, a_s.reshape(N), a_d.reshape(N), maxa16, z)
  return _combine(s1p, s0p.reshape(NC, N, 1), gat_b.reshape(1, OUT))


# speedup vs baseline: 45.8806x; 1.0546x over previous
import functools

import jax
import jax.numpy as jnp
from jax import lax
from jax.experimental import pallas as pl
from jax.experimental.pallas import tpu as pltpu
from jax.experimental.pallas import tpu_sc as plsc

N = 10000
E = 320000
HID = 128
OUT = 128

NC = 2
NS = 16
L = 16
NW = NC * NS
EPW = E // NW
K = 80
NCH = EPW // K
CPG = 25
GPT = NCH // CPG
NRC = N // K
RRK = (NRC + NS - 1) // NS

BLK = 1000
G = N // BLK


def _dense_body(x_ref, w1_ref, b1_ref, w2_ref, as_w_ref, ad_w_ref,
                z_ref, asrc_ref, adst_ref, pmax_ref):
  x = x_ref[...]
  h = lax.dot_general(x, w1_ref[...], (((1,), (1,)), ((), ())),
                      preferred_element_type=jnp.float32)
  h = jnp.maximum(h + b1_ref[...], 0.0)
  z = jnp.dot(h, w2_ref[...], preferred_element_type=jnp.float32)
  z_ref[...] = z
  a_s = jnp.sum(z * as_w_ref[...], axis=1, keepdims=True)
  a_d = jnp.sum(z * ad_w_ref[...], axis=1, keepdims=True)
  asrc_ref[...] = a_s
  adst_ref[...] = a_d
  i = pl.program_id(0)
  pmax_ref[pl.ds(i, 1), :] = jnp.max(a_s).reshape(1, 1)


_dense = pl.pallas_call(
    _dense_body,
    grid=(G,),
    in_specs=[
        pl.BlockSpec((BLK, HID), lambda i: (i, 0)),
        pl.BlockSpec((HID, HID), lambda i: (0, 0)),
        pl.BlockSpec((1, HID), lambda i: (0, 0)),
        pl.BlockSpec((HID, OUT), lambda i: (0, 0)),
        pl.BlockSpec((1, OUT), lambda i: (0, 0)),
        pl.BlockSpec((1, OUT), lambda i: (0, 0)),
    ],
    out_specs=[
        pl.BlockSpec((BLK, OUT), lambda i: (i, 0)),
        pl.BlockSpec((BLK, 1), lambda i: (i, 0)),
        pl.BlockSpec((BLK, 1), lambda i: (i, 0)),
        pl.BlockSpec((G, 1), lambda i: (0, 0)),
    ],
    out_shape=[
        jax.ShapeDtypeStruct((N, OUT), jnp.float32),
        jax.ShapeDtypeStruct((N, 1), jnp.float32),
        jax.ShapeDtypeStruct((N, 1), jnp.float32),
        jax.ShapeDtypeStruct((G, 1), jnp.float32),
    ],
)


def _sc_body(edge_hbm, asrc_hbm, adst_hbm, maxa_hbm, z_hbm,
             s1_hbm, s0_hbm,
             asrc_v, adst_v, maxa_v, sidx_g, didx_g, w0, w1, rows0, rows1,
             s0_sh, s1_sh, semg0, semg1, sems1_0, sems1_1, sems0_0, sems0_1):
  c = lax.axis_index("c")
  s = lax.axis_index("s")
  wid = c * NS + s

  pltpu.sync_copy(asrc_hbm, asrc_v)
  pltpu.sync_copy(adst_hbm, adst_v)
  pltpu.sync_copy(maxa_hbm, maxa_v)

  zv = jnp.zeros((L,), jnp.float32)

  for q in range(K // L):
    w0[pl.ds(q * L, L)] = zv

  def _zrow(r, carry):
    for q in range(OUT // L):
      rows0[r, pl.ds(q * L, L)] = zv
    return carry

  lax.fori_loop(0, K, _zrow, 0)

  for k in range(RRK):
    j = s + NS * k

    @pl.when(j < NRC)
    def _zero_acc():
      pltpu.sync_copy(rows0, s1_sh.at[pl.ds(j * K, K)])
      pltpu.sync_copy(w0, s0_sh.at[pl.ds(j * K, K)])

  plsc.subcore_barrier()

  maxa = maxa_v[...]

  def _do_chunk(j, rows_b, w_b, semg_b, sems1_b, sems0_b,
                rows_n, w_n, semg_n, sems1_n, sems0_n):
    for q in range(K // L):
      si = sidx_g[j, pl.ds(q * L, L)]
      di = didx_g[j, pl.ds(q * L, L)]
      gs = plsc.load_gather(asrc_v, (si,))
      gd = plsc.load_gather(adst_v, (di,))
      e = gs + gd
      e = jnp.where(e >= 0.0, e, 0.2 * e)
      m = maxa + gd
      m = jnp.where(m >= 0.0, m, 0.2 * m)
      w_b[pl.ds(q * L, L)] = jnp.exp(e - m)
    pltpu.make_async_copy(z_hbm.at[sidx_g.at[j]], rows_b, semg_b).wait()

    @pl.when(j >= 1)
    def _drain_prev():
      pltpu.make_async_copy(rows_n, s1_sh.at[didx_g.at[j - 1]], sems1_n).wait()
      pltpu.make_async_copy(w_n, s0_sh.at[didx_g.at[j - 1]], sems0_n).wait()

    @pl.when(j + 1 < CPG)
    def _fire_next():
      pltpu.async_copy(z_hbm.at[sidx_g.at[j + 1]], rows_n, semg_n)

    def _scale(g, carry2):
      wv = w_b[pl.ds(pl.multiple_of(g * L, 8), L)]
      for t in range(L):
        wj = wv[t]
        r = g * L + t
        for q in range(OUT // L):
          rows_b[r, pl.ds(q * L, L)] = rows_b[r, pl.ds(q * L, L)] * wj
      return carry2

    lax.fori_loop(0, K // L, _scale, 0)
    pltpu.async_copy(rows_b, s1_sh.at[didx_g.at[j]], sems1_b, add=True)
    pltpu.async_copy(w_b, s0_sh.at[didx_g.at[j]], sems0_b, add=True)

  def _group(g, carry):
    gb = wid * GPT + g
    d1 = pltpu.async_copy(edge_hbm.at[0, gb], sidx_g, semg0)
    d2 = pltpu.async_copy(edge_hbm.at[1, gb], didx_g, semg1)
    d1.wait()
    d2.wait()
    pltpu.async_copy(z_hbm.at[sidx_g.at[0]], rows0, semg0)

    def _pair(t, carry2):
      _do_chunk(2 * t, rows0, w0, semg0, sems1_0, sems0_0,
                rows1, w1, semg1, sems1_1, sems0_1)

      @pl.when(2 * t + 1 < CPG)
      def _odd():
        _do_chunk(2 * t + 1, rows1, w1, semg1, sems1_1, sems0_1,
                  rows0, w0, semg0, sems1_0, sems0_0)

      return carry2

    lax.fori_loop(0, (CPG + 1) // 2, _pair, 0)
    pltpu.make_async_copy(rows0, s1_sh.at[didx_g.at[CPG - 1]], sems1_0).wait()
    pltpu.make_async_copy(w0, s0_sh.at[didx_g.at[CPG - 1]], sems0_0).wait()
    return carry

  lax.fori_loop(0, GPT, _group, 0)
  plsc.subcore_barrier()

  for k in range(RRK):
    j = s + NS * k

    @pl.when(j < NRC)
    def _out_acc():
      r0 = pl.multiple_of(j * K, 8)
      pltpu.sync_copy(s1_sh.at[pl.ds(r0, K)], rows0)
      pltpu.sync_copy(rows0, s1_hbm.at[c, pl.ds(r0, K)])
      pltpu.sync_copy(s0_sh.at[pl.ds(r0, K)], w0)
      pltpu.sync_copy(w0, s0_hbm.at[pl.ds(pl.multiple_of(c * N + r0, 8), K)])


_edge = functools.partial(
    pl.kernel,
    out_type=(
        jax.ShapeDtypeStruct((NC, N, OUT), jnp.float32),
        jax.ShapeDtypeStruct((NC * N,), jnp.float32),
    ),
    mesh=plsc.VectorSubcoreMesh(core_axis_name="c", subcore_axis_name="s"),
    scratch_types=[
        pltpu.VMEM((N,), jnp.float32),
        pltpu.VMEM((N,), jnp.float32),
        pltpu.VMEM((L,), jnp.float32),
        pltpu.VMEM((CPG, K), jnp.int32),
        pltpu.VMEM((CPG, K), jnp.int32),
        pltpu.VMEM((K,), jnp.float32),
        pltpu.VMEM((K,), jnp.float32),
        pltpu.VMEM((K, OUT), jnp.float32),
        pltpu.VMEM((K, OUT), jnp.float32),
        pltpu.VMEM_SHARED((N,), jnp.float32),
        pltpu.VMEM_SHARED((N, OUT), jnp.float32),
        pltpu.SemaphoreType.DMA,
        pltpu.SemaphoreType.DMA,
        pltpu.SemaphoreType.DMA,
        pltpu.SemaphoreType.DMA,
        pltpu.SemaphoreType.DMA,
        pltpu.SemaphoreType.DMA,
    ],
    compiler_params=pltpu.CompilerParams(needs_layout_passes=False),
)(_sc_body)


def _combine_body(s1_ref, s0_ref, b_ref, o_ref):
  s1 = s1_ref[0] + s1_ref[1]
  s0 = jnp.sum(s0_ref[...], axis=0)
  o_ref[...] = jnp.maximum(s1 / (s0 + 1e-16) + b_ref[...], 0.0)


_combine = pl.pallas_call(
    _combine_body,
    grid=(G,),
    in_specs=[
        pl.BlockSpec((NC, BLK, OUT), lambda i: (0, i, 0)),
        pl.BlockSpec((NC, BLK, 1), lambda i: (0, i, 0)),
        pl.BlockSpec((1, OUT), lambda i: (0, 0)),
    ],
    out_specs=pl.BlockSpec((BLK, OUT), lambda i: (i, 0)),
    out_shape=jax.ShapeDtypeStruct((N, OUT), jnp.float32),
)


@jax.jit
def kernel(x, edge_index, fc1_W, fc1_b, gat_W, att_src, att_dst, gat_b):
  z, a_s, a_d, pmax = _dense(
      x, fc1_W, fc1_b.reshape(1, HID), gat_W,
      att_src.reshape(1, OUT), att_dst.reshape(1, OUT))
  maxa16 = jnp.full((L,), jnp.max(pmax), jnp.float32)
  edge3 = edge_index.reshape(2, NW * GPT, CPG, K)
  s1p, s0p = _edge(edge3, a_s.reshape(N), a_d.reshape(N), maxa16, z)
  return _combine(s1p, s0p.reshape(NC, N, 1), gat_b.reshape(1, OUT))
